# baseline (speedup 1.0000x reference)
import functools

import jax
import jax.numpy as jnp
from jax import lax
from jax.experimental import pallas as pl
from jax.experimental.pallas import tpu as pltpu
from jax.experimental.pallas import tpu_sc as plsc

VOCAB = 30522
D = 768
NCLS = 2
B = 4096
L = 200

BLK = 3072
VP = 30720
BIAS_COL = 30528

NC = 2
NS = 16
NW = NC * NS
SEQ_PER_W = B // NW
GROUPS = SEQ_PER_W // 16


def _tw_body(w_ref, tbl_ref, b_ref, out_ref):
    res = lax.dot_general(
        w_ref[...], tbl_ref[...],
        dimension_numbers=(((1,), (1,)), ((), ())),
        preferred_element_type=jnp.float32,
    )
    h0 = lax.bitcast_convert_type(
        res[0, :].astype(jnp.bfloat16), jnp.uint16).astype(jnp.uint32)
    h1 = lax.bitcast_convert_type(
        res[1, :].astype(jnp.bfloat16), jnp.uint16).astype(jnp.uint32)
    packed = lax.bitcast_convert_type(h0 | (h1 << 16), jnp.int32)

    @pl.when(pl.program_id(0) == BIAS_COL // BLK)
    def _():
        pos = lax.iota(jnp.int32, BLK)
        off = BIAS_COL % BLK
        bw0 = lax.bitcast_convert_type(b_ref[0, 0], jnp.int32)
        bw1 = lax.bitcast_convert_type(b_ref[1, 0], jnp.int32)
        out_ref[...] = jnp.where(
            pos == off, bw0, jnp.where(pos == off + 1, bw1, packed))

    @pl.when(pl.program_id(0) != BIAS_COL // BLK)
    def _():
        out_ref[...] = packed


def _compute_tw(table, W, b2):
    return pl.pallas_call(
        _tw_body,
        grid=(VP // BLK,),
        in_specs=[
            pl.BlockSpec((NCLS, D), lambda i: (0, 0)),
            pl.BlockSpec((BLK, D), lambda i: (i, 0)),
            pl.BlockSpec((NCLS, 1), lambda i: (0, 0)),
        ],
        out_specs=pl.BlockSpec((BLK,), lambda i: (i,)),
        out_shape=jax.ShapeDtypeStruct((VP,), jnp.int32),
    )(W, table, b2)


def _sc_kernel(twp_hbm, ids_hbm, out0_hbm, out1_hbm,
               twp_v, ids_v, out0_v, out1_v, sem_a, sem_b):
    wid = lax.axis_index("s") * NC + lax.axis_index("c")
    base = wid * SEQ_PER_W

    cp_tw = pltpu.async_copy(twp_hbm, twp_v, sem_a)
    cp_ids = pltpu.async_copy(ids_hbm.at[wid], ids_v, sem_b)
    cp_tw.wait()
    cp_ids.wait()

    zero = jnp.zeros((16,), jnp.float32)

    def body(t, accs):
        new = []
        for g in range(GROUPS):
            idx = ids_v[t, pl.ds(g * 16, 16)]
            pw = plsc.load_gather(twp_v, [idx])
            bb = plsc.bitcast(pw, jnp.bfloat16)
            v0, v1 = plsc.unpack(bb, format=plsc.PackFormat.INTERLEAVED)
            new.append(accs[2 * g] + v0)
            new.append(accs[2 * g + 1] + v1)
        return tuple(new)

    accs = lax.fori_loop(0, L, body, (zero,) * (2 * GROUPS), unroll=4)

    inv_l = jnp.float32(1.0 / L)
    bvec = plsc.bitcast(twp_v[pl.ds(BIAS_COL, 16)], jnp.float32)
    b0 = bvec[0]
    b1 = bvec[1]
    for g in range(GROUPS):
        out0_v[pl.ds(g * 16, 16)] = accs[2 * g] * inv_l + b0
        out1_v[pl.ds(g * 16, 16)] = accs[2 * g + 1] * inv_l + b1

    pltpu.sync_copy(out0_v, out0_hbm.at[pl.ds(base, SEQ_PER_W)])
    pltpu.sync_copy(out1_v, out1_hbm.at[pl.ds(base, SEQ_PER_W)])


def _pool_logits(twp, ids):
    mesh = plsc.VectorSubcoreMesh(core_axis_name="c", subcore_axis_name="s")
    f = functools.partial(
        pl.kernel,
        mesh=mesh,
        out_type=(
            jax.ShapeDtypeStruct((B,), jnp.float32),
            jax.ShapeDtypeStruct((B,), jnp.float32),
        ),
        scratch_types=[
            pltpu.VMEM((VP,), jnp.int32),
            pltpu.VMEM((L, SEQ_PER_W), jnp.int32),
            pltpu.VMEM((SEQ_PER_W,), jnp.float32),
            pltpu.VMEM((SEQ_PER_W,), jnp.float32),
            pltpu.SemaphoreType.DMA,
            pltpu.SemaphoreType.DMA,
        ],
        compiler_params=pltpu.CompilerParams(needs_layout_passes=False),
    )(_sc_kernel)
    return f(twp, ids)


def kernel(input_ids, table, W, b):
    b2 = b.astype(jnp.float32).reshape(NCLS, 1)
    twp = _compute_tw(table, W, b2)
    ids = input_ids.astype(jnp.int32).reshape(NW, SEQ_PER_W, L).transpose(0, 2, 1)
    out0, out1 = _pool_logits(twp, ids)
    return jnp.stack([out0, out1], axis=-1)


# speedup vs baseline: 1.0279x; 1.0279x over previous
import functools

import jax
import jax.numpy as jnp
from jax import lax
from jax.experimental import pallas as pl
from jax.experimental.pallas import tpu as pltpu
from jax.experimental.pallas import tpu_sc as plsc

VOCAB = 30522
D = 768
NCLS = 2
B = 4096
L = 200

BLK = 3072
VP = 30720
BIAS_COL = 30528

NC = 2
NS = 16
NW = NC * NS
SEQ_PER_W = B // NW
GROUPS = SEQ_PER_W // 16


def _tw_body(w_ref, tbl_ref, b_ref, out_ref):
    res = lax.dot_general(
        w_ref[...], tbl_ref[...],
        dimension_numbers=(((1,), (1,)), ((), ())),
        preferred_element_type=jnp.float32,
    )
    h0 = lax.bitcast_convert_type(
        res[0, :].astype(jnp.bfloat16), jnp.uint16).astype(jnp.uint32)
    h1 = lax.bitcast_convert_type(
        res[1, :].astype(jnp.bfloat16), jnp.uint16).astype(jnp.uint32)
    packed = lax.bitcast_convert_type(h0 | (h1 << 16), jnp.int32)

    @pl.when(pl.program_id(0) == BIAS_COL // BLK)
    def _():
        pos = lax.iota(jnp.int32, BLK)
        off = BIAS_COL % BLK
        bw0 = lax.bitcast_convert_type(b_ref[0, 0], jnp.int32)
        bw1 = lax.bitcast_convert_type(b_ref[1, 0], jnp.int32)
        out_ref[...] = jnp.where(
            pos == off, bw0, jnp.where(pos == off + 1, bw1, packed))

    @pl.when(pl.program_id(0) != BIAS_COL // BLK)
    def _():
        out_ref[...] = packed


def _compute_tw(table, W, b2):
    return pl.pallas_call(
        _tw_body,
        grid=(VP // BLK,),
        in_specs=[
            pl.BlockSpec((NCLS, D), lambda i: (0, 0)),
            pl.BlockSpec((BLK, D), lambda i: (i, 0)),
            pl.BlockSpec((NCLS, 1), lambda i: (0, 0)),
        ],
        out_specs=pl.BlockSpec((BLK,), lambda i: (i,)),
        out_shape=jax.ShapeDtypeStruct((VP,), jnp.int32),
    )(W, table, b2)


def _sc_kernel(twp_hbm, ids_hbm, out0_hbm, out1_hbm,
               twp_v, ids_v, out0_v, out1_v, sem_a, sem_b):
    wid = lax.axis_index("s") * NC + lax.axis_index("c")
    base = wid * SEQ_PER_W

    cp_tw = pltpu.async_copy(twp_hbm, twp_v, sem_a)
    cp_ids = pltpu.async_copy(
        ids_hbm.at[:, pl.ds(base, SEQ_PER_W)], ids_v, sem_b)
    cp_tw.wait()
    cp_ids.wait()

    zero = jnp.zeros((16,), jnp.float32)

    def body(t, accs):
        new = []
        for g in range(GROUPS):
            idx = ids_v[t, pl.ds(g * 16, 16)]
            pw = plsc.load_gather(twp_v, [idx])
            bb = plsc.bitcast(pw, jnp.bfloat16)
            v0, v1 = plsc.unpack(bb, format=plsc.PackFormat.INTERLEAVED)
            new.append(accs[2 * g] + v0)
            new.append(accs[2 * g + 1] + v1)
        return tuple(new)

    accs = lax.fori_loop(0, L, body, (zero,) * (2 * GROUPS), unroll=2)

    inv_l = jnp.float32(1.0 / L)
    bvec = plsc.bitcast(twp_v[pl.ds(BIAS_COL, 16)], jnp.float32)
    b0 = bvec[0]
    b1 = bvec[1]
    for g in range(GROUPS):
        out0_v[pl.ds(g * 16, 16)] = accs[2 * g] * inv_l + b0
        out1_v[pl.ds(g * 16, 16)] = accs[2 * g + 1] * inv_l + b1

    pltpu.sync_copy(out0_v, out0_hbm.at[pl.ds(base, SEQ_PER_W)])
    pltpu.sync_copy(out1_v, out1_hbm.at[pl.ds(base, SEQ_PER_W)])


def _pool_logits(twp, ids):
    mesh = plsc.VectorSubcoreMesh(core_axis_name="c", subcore_axis_name="s")
    f = functools.partial(
        pl.kernel,
        mesh=mesh,
        out_type=(
            jax.ShapeDtypeStruct((B,), jnp.float32),
            jax.ShapeDtypeStruct((B,), jnp.float32),
        ),
        scratch_types=[
            pltpu.VMEM((VP,), jnp.int32),
            pltpu.VMEM((L, SEQ_PER_W), jnp.int32),
            pltpu.VMEM((SEQ_PER_W,), jnp.float32),
            pltpu.VMEM((SEQ_PER_W,), jnp.float32),
            pltpu.SemaphoreType.DMA,
            pltpu.SemaphoreType.DMA,
        ],
        compiler_params=pltpu.CompilerParams(needs_layout_passes=False),
    )(_sc_kernel)
    return f(twp, ids)


def kernel(input_ids, table, W, b):
    b2 = b.astype(jnp.float32).reshape(NCLS, 1)
    twp = _compute_tw(table, W, b2)
    ids = input_ids.astype(jnp.int32).T
    out0, out1 = _pool_logits(twp, ids)
    return jnp.stack([out0, out1], axis=-1)


# speedup vs baseline: 1.1100x; 1.0799x over previous
import functools

import jax
import jax.numpy as jnp
from jax import lax
from jax.experimental import pallas as pl
from jax.experimental.pallas import tpu as pltpu
from jax.experimental.pallas import tpu_sc as plsc

VOCAB = 30522
D = 768
NCLS = 2
B = 4096
L = 200

BLK = 3072
VP = 30720
BIAS_COL = 30528

NC = 2
NS = 16
NW = NC * NS
SEQ_PER_W = B // NW
GROUPS = SEQ_PER_W // 16


def _tw_body(w_ref, tbl_ref, b_ref, out_ref):
    res = lax.dot_general(
        w_ref[...], tbl_ref[...],
        dimension_numbers=(((1,), (1,)), ((), ())),
        preferred_element_type=jnp.float32,
    )
    h0 = lax.bitcast_convert_type(
        res[0, :].astype(jnp.bfloat16), jnp.uint16).astype(jnp.uint32)
    h1 = lax.bitcast_convert_type(
        res[1, :].astype(jnp.bfloat16), jnp.uint16).astype(jnp.uint32)
    packed = lax.bitcast_convert_type(h0 | (h1 << 16), jnp.int32)

    @pl.when(pl.program_id(0) == BIAS_COL // BLK)
    def _():
        pos = lax.iota(jnp.int32, BLK)
        off = BIAS_COL % BLK
        bw0 = lax.bitcast_convert_type(b_ref[0, 0], jnp.int32)
        bw1 = lax.bitcast_convert_type(b_ref[1, 0], jnp.int32)
        out_ref[...] = jnp.where(
            pos == off, bw0, jnp.where(pos == off + 1, bw1, packed))

    @pl.when(pl.program_id(0) != BIAS_COL // BLK)
    def _():
        out_ref[...] = packed


def _compute_tw(table, W, b2):
    return pl.pallas_call(
        _tw_body,
        grid=(VP // BLK,),
        in_specs=[
            pl.BlockSpec((NCLS, D), lambda i: (0, 0)),
            pl.BlockSpec((BLK, D), lambda i: (i, 0)),
            pl.BlockSpec((NCLS, 1), lambda i: (0, 0)),
        ],
        out_specs=pl.BlockSpec((BLK,), lambda i: (i,)),
        out_shape=jax.ShapeDtypeStruct((VP,), jnp.int32),
    )(W, table, b2)


def _sc_kernel(twp_hbm, ids_hbm, out0_hbm, out1_hbm,
               twp_v, ids_v, out0_v, out1_v, twp_s, sem_a, sem_b):
    sid = lax.axis_index("s")
    wid = sid * NC + lax.axis_index("c")
    base = wid * SEQ_PER_W

    cp_ids = pltpu.async_copy(
        ids_hbm.at[:, pl.ds(base, SEQ_PER_W)], ids_v, sem_b)

    @pl.when(sid == 0)
    def _():
        pltpu.sync_copy(twp_hbm, twp_s)

    plsc.subcore_barrier()
    cp_tw = pltpu.async_copy(twp_s, twp_v, sem_a)
    cp_tw.wait()
    cp_ids.wait()

    zero = jnp.zeros((16,), jnp.float32)

    def body(t, accs):
        new = []
        for g in range(GROUPS):
            idx = ids_v[t, pl.ds(g * 16, 16)]
            pw = plsc.load_gather(twp_v, [idx])
            bb = plsc.bitcast(pw, jnp.bfloat16)
            v0, v1 = plsc.unpack(bb, format=plsc.PackFormat.INTERLEAVED)
            new.append(accs[2 * g] + v0)
            new.append(accs[2 * g + 1] + v1)
        return tuple(new)

    accs = lax.fori_loop(0, L, body, (zero,) * (2 * GROUPS), unroll=2)

    inv_l = jnp.float32(1.0 / L)
    bvec = plsc.bitcast(twp_v[pl.ds(BIAS_COL, 16)], jnp.float32)
    b0 = bvec[0]
    b1 = bvec[1]
    for g in range(GROUPS):
        out0_v[pl.ds(g * 16, 16)] = accs[2 * g] * inv_l + b0
        out1_v[pl.ds(g * 16, 16)] = accs[2 * g + 1] * inv_l + b1

    pltpu.sync_copy(out0_v, out0_hbm.at[pl.ds(base, SEQ_PER_W)])
    pltpu.sync_copy(out1_v, out1_hbm.at[pl.ds(base, SEQ_PER_W)])


def _pool_logits(twp, ids):
    mesh = plsc.VectorSubcoreMesh(core_axis_name="c", subcore_axis_name="s")
    f = functools.partial(
        pl.kernel,
        mesh=mesh,
        out_type=(
            jax.ShapeDtypeStruct((B,), jnp.float32),
            jax.ShapeDtypeStruct((B,), jnp.float32),
        ),
        scratch_types=[
            pltpu.VMEM((VP,), jnp.int32),
            pltpu.VMEM((L, SEQ_PER_W), jnp.int32),
            pltpu.VMEM((SEQ_PER_W,), jnp.float32),
            pltpu.VMEM((SEQ_PER_W,), jnp.float32),
            pltpu.VMEM_SHARED((VP,), jnp.int32),
            pltpu.SemaphoreType.DMA,
            pltpu.SemaphoreType.DMA,
        ],
        compiler_params=pltpu.CompilerParams(needs_layout_passes=False),
    )(_sc_kernel)
    return f(twp, ids)


def kernel(input_ids, table, W, b):
    b2 = b.astype(jnp.float32).reshape(NCLS, 1)
    twp = _compute_tw(table, W, b2)
    ids = input_ids.astype(jnp.int32).T
    out0, out1 = _pool_logits(twp, ids)
    return jnp.stack([out0, out1], axis=-1)


# speedup vs baseline: 1.1115x; 1.0014x over previous
import functools

import jax
import jax.numpy as jnp
from jax import lax
from jax.experimental import pallas as pl
from jax.experimental.pallas import tpu as pltpu
from jax.experimental.pallas import tpu_sc as plsc

VOCAB = 30522
D = 768
NCLS = 2
B = 4096
L = 200

BLK = 3072
VP = 30720
BIAS_COL = 30528

NC = 2
NS = 16
NW = NC * NS
SEQ_PER_W = B // NW
GROUPS = SEQ_PER_W // 16


def _tw_body(w_ref, tbl_lo_ref, tbl_hi_ref, b_ref, out_ref):
    w = w_ref[...]
    res = lax.dot_general(
        w[:, : D // 2], tbl_lo_ref[...],
        dimension_numbers=(((1,), (1,)), ((), ())),
        preferred_element_type=jnp.float32,
    ) + lax.dot_general(
        w[:, D // 2:], tbl_hi_ref[...],
        dimension_numbers=(((1,), (1,)), ((), ())),
        preferred_element_type=jnp.float32,
    )
    h0 = lax.bitcast_convert_type(
        res[0, :].astype(jnp.bfloat16), jnp.uint16).astype(jnp.uint32)
    h1 = lax.bitcast_convert_type(
        res[1, :].astype(jnp.bfloat16), jnp.uint16).astype(jnp.uint32)
    packed = lax.bitcast_convert_type(h0 | (h1 << 16), jnp.int32)

    @pl.when(pl.program_id(0) == BIAS_COL // BLK)
    def _():
        pos = lax.iota(jnp.int32, BLK)
        off = BIAS_COL % BLK
        bw0 = lax.bitcast_convert_type(b_ref[0, 0], jnp.int32)
        bw1 = lax.bitcast_convert_type(b_ref[1, 0], jnp.int32)
        out_ref[...] = jnp.where(
            pos == off, bw0, jnp.where(pos == off + 1, bw1, packed))

    @pl.when(pl.program_id(0) != BIAS_COL // BLK)
    def _():
        out_ref[...] = packed


def _compute_tw(table, W, b2):
    return pl.pallas_call(
        _tw_body,
        grid=(VP // BLK,),
        in_specs=[
            pl.BlockSpec((NCLS, D), lambda i: (0, 0)),
            pl.BlockSpec((BLK, D // 2), lambda i: (i, 0)),
            pl.BlockSpec((BLK, D // 2), lambda i: (i, 1)),
            pl.BlockSpec((NCLS, 1), lambda i: (0, 0)),
        ],
        out_specs=pl.BlockSpec((BLK,), lambda i: (i,)),
        out_shape=jax.ShapeDtypeStruct((VP,), jnp.int32),
    )(W, table, table, b2)


def _sc_kernel(twp_hbm, ids_hbm, out0_hbm, out1_hbm,
               twp_v, ids_v, out0_v, out1_v, twp_s, sem_a, sem_b):
    sid = lax.axis_index("s")
    wid = sid * NC + lax.axis_index("c")
    base = wid * SEQ_PER_W

    cp_ids = pltpu.async_copy(
        ids_hbm.at[:, pl.ds(base, SEQ_PER_W)], ids_v, sem_b)

    @pl.when(sid == 0)
    def _():
        pltpu.sync_copy(twp_hbm, twp_s)

    plsc.subcore_barrier()
    cp_tw = pltpu.async_copy(twp_s, twp_v, sem_a)
    cp_tw.wait()
    cp_ids.wait()

    zero = jnp.zeros((16,), jnp.float32)

    def body(t, accs):
        new = []
        for g in range(GROUPS):
            idx = ids_v[t, pl.ds(g * 16, 16)]
            pw = plsc.load_gather(twp_v, [idx])
            bb = plsc.bitcast(pw, jnp.bfloat16)
            v0, v1 = plsc.unpack(bb, format=plsc.PackFormat.INTERLEAVED)
            new.append(accs[2 * g] + v0)
            new.append(accs[2 * g + 1] + v1)
        return tuple(new)

    accs = lax.fori_loop(0, L, body, (zero,) * (2 * GROUPS), unroll=2)

    inv_l = jnp.float32(1.0 / L)
    bvec = plsc.bitcast(twp_v[pl.ds(BIAS_COL, 16)], jnp.float32)
    b0 = bvec[0]
    b1 = bvec[1]
    for g in range(GROUPS):
        out0_v[pl.ds(g * 16, 16)] = accs[2 * g] * inv_l + b0
        out1_v[pl.ds(g * 16, 16)] = accs[2 * g + 1] * inv_l + b1

    pltpu.sync_copy(out0_v, out0_hbm.at[pl.ds(base, SEQ_PER_W)])
    pltpu.sync_copy(out1_v, out1_hbm.at[pl.ds(base, SEQ_PER_W)])


def _pool_logits(twp, ids):
    mesh = plsc.VectorSubcoreMesh(core_axis_name="c", subcore_axis_name="s")
    f = functools.partial(
        pl.kernel,
        mesh=mesh,
        out_type=(
            jax.ShapeDtypeStruct((B,), jnp.float32),
            jax.ShapeDtypeStruct((B,), jnp.float32),
        ),
        scratch_types=[
            pltpu.VMEM((VP,), jnp.int32),
            pltpu.VMEM((L, SEQ_PER_W), jnp.int32),
            pltpu.VMEM((SEQ_PER_W,), jnp.float32),
            pltpu.VMEM((SEQ_PER_W,), jnp.float32),
            pltpu.VMEM_SHARED((VP,), jnp.int32),
            pltpu.SemaphoreType.DMA,
            pltpu.SemaphoreType.DMA,
        ],
        compiler_params=pltpu.CompilerParams(needs_layout_passes=False),
    )(_sc_kernel)
    return f(twp, ids)


def kernel(input_ids, table, W, b):
    b2 = b.astype(jnp.float32).reshape(NCLS, 1)
    twp = _compute_tw(table, W, b2)
    ids = input_ids.astype(jnp.int32).T
    out0, out1 = _pool_logits(twp, ids)
    return jnp.stack([out0, out1], axis=-1)


# speedup vs baseline: 1.1130x; 1.0013x over previous
import functools

import jax
import jax.numpy as jnp
from jax import lax
from jax.experimental import pallas as pl
from jax.experimental.pallas import tpu as pltpu
from jax.experimental.pallas import tpu_sc as plsc

VOCAB = 30522
D = 768
NCLS = 2
B = 4096
L = 200

BLK = 3072
VP = 30720
BIAS_COL = 30528

NC = 2
NS = 16
NW = NC * NS
SEQ_PER_W = B // NW
GROUPS = SEQ_PER_W // 16


def _tw_body(w_ref, tbl_ref, b_ref, out_ref):
    res = lax.dot_general(
        w_ref[...], tbl_ref[...],
        dimension_numbers=(((1,), (1,)), ((), ())),
        preferred_element_type=jnp.float32,
    )
    h0 = lax.bitcast_convert_type(
        res[0, :].astype(jnp.bfloat16), jnp.uint16).astype(jnp.uint32)
    h1 = lax.bitcast_convert_type(
        res[1, :].astype(jnp.bfloat16), jnp.uint16).astype(jnp.uint32)
    packed = lax.bitcast_convert_type(h0 | (h1 << 16), jnp.int32)

    @pl.when(pl.program_id(0) == BIAS_COL // BLK)
    def _():
        pos = lax.iota(jnp.int32, BLK)
        off = BIAS_COL % BLK
        bw0 = lax.bitcast_convert_type(b_ref[0, 0], jnp.int32)
        bw1 = lax.bitcast_convert_type(b_ref[1, 0], jnp.int32)
        out_ref[...] = jnp.where(
            pos == off, bw0, jnp.where(pos == off + 1, bw1, packed))

    @pl.when(pl.program_id(0) != BIAS_COL // BLK)
    def _():
        out_ref[...] = packed


def _compute_tw(table, W, b2):
    return pl.pallas_call(
        _tw_body,
        grid=(VP // BLK,),
        in_specs=[
            pl.BlockSpec((NCLS, D), lambda i: (0, 0)),
            pl.BlockSpec((BLK, D), lambda i: (i, 0)),
            pl.BlockSpec((NCLS, 1), lambda i: (0, 0)),
        ],
        out_specs=pl.BlockSpec((BLK,), lambda i: (i,)),
        out_shape=jax.ShapeDtypeStruct((VP,), jnp.int32),
    )(W, table, b2)


def _sc_kernel(twp_hbm, ids_hbm, out0_hbm, out1_hbm,
               twp_v, ids_v, out0_v, out1_v, twp_s, sem_a, sem_b):
    sid = lax.axis_index("s")
    wid = sid * NC + lax.axis_index("c")
    base = wid * SEQ_PER_W

    cp_ids = pltpu.async_copy(
        ids_hbm.at[:, pl.ds(base, SEQ_PER_W)], ids_v, sem_b)

    @pl.when(sid == 0)
    def _():
        pltpu.sync_copy(twp_hbm, twp_s)

    plsc.subcore_barrier()
    cp_tw = pltpu.async_copy(twp_s, twp_v, sem_a)
    cp_tw.wait()
    cp_ids.wait()

    zero = jnp.zeros((16,), jnp.float32)

    def body(t, accs):
        new = []
        for g in range(GROUPS):
            idx = ids_v[t, pl.ds(g * 16, 16)]
            pw = plsc.load_gather(twp_v, [idx])
            bb = plsc.bitcast(pw, jnp.bfloat16)
            v0, v1 = plsc.unpack(bb, format=plsc.PackFormat.INTERLEAVED)
            new.append(accs[2 * g] + v0)
            new.append(accs[2 * g + 1] + v1)
        return tuple(new)

    accs = lax.fori_loop(0, L, body, (zero,) * (2 * GROUPS), unroll=2)

    inv_l = jnp.float32(1.0 / L)
    bvec = plsc.bitcast(twp_v[pl.ds(BIAS_COL, 16)], jnp.float32)
    b0 = bvec[0]
    b1 = bvec[1]
    for g in range(GROUPS):
        out0_v[pl.ds(g * 16, 16)] = accs[2 * g] * inv_l + b0
        out1_v[pl.ds(g * 16, 16)] = accs[2 * g + 1] * inv_l + b1

    pltpu.sync_copy(out0_v, out0_hbm.at[pl.ds(base, SEQ_PER_W)])
    pltpu.sync_copy(out1_v, out1_hbm.at[pl.ds(base, SEQ_PER_W)])


def _pool_logits(twp, ids):
    mesh = plsc.VectorSubcoreMesh(core_axis_name="c", subcore_axis_name="s")
    f = functools.partial(
        pl.kernel,
        mesh=mesh,
        out_type=(
            jax.ShapeDtypeStruct((B,), jnp.float32),
            jax.ShapeDtypeStruct((B,), jnp.float32),
        ),
        scratch_types=[
            pltpu.VMEM((VP,), jnp.int32),
            pltpu.VMEM((L, SEQ_PER_W), jnp.int32),
            pltpu.VMEM((SEQ_PER_W,), jnp.float32),
            pltpu.VMEM((SEQ_PER_W,), jnp.float32),
            pltpu.VMEM_SHARED((VP,), jnp.int32),
            pltpu.SemaphoreType.DMA,
            pltpu.SemaphoreType.DMA,
        ],
        compiler_params=pltpu.CompilerParams(needs_layout_passes=False),
    )(_sc_kernel)
    return f(twp, ids)


def kernel(input_ids, table, W, b):
    b2 = b.astype(jnp.float32).reshape(NCLS, 1)
    twp = _compute_tw(table, W, b2)
    ids = input_ids.astype(jnp.int32).T
    out0, out1 = _pool_logits(twp, ids)
    return jnp.stack([out0, out1], axis=-1)
